# baseline (speedup 1.0000x reference)
import jax
import jax.numpy as jnp
from jax.experimental import pallas as pl
from jax.experimental.pallas import tpu as pltpu

_N_CHUNKS = 8


def _copy_scatter_kernel(x_ref, o_ref, row0, rowl, sems, row_sem):
    n_rows, n_cols = x_ref.shape
    chunk = n_rows // _N_CHUNKS

    r0 = pltpu.make_async_copy(x_ref.at[0:1, :], row0, row_sem)
    r0.start()
    rl = pltpu.make_async_copy(x_ref.at[n_rows - 1 : n_rows, :], rowl, row_sem)
    rl.start()

    copies = []
    for k in range(_N_CHUNKS):
        c = pltpu.make_async_copy(
            x_ref.at[pl.ds(k * chunk, chunk), :],
            o_ref.at[pl.ds(k * chunk, chunk), :],
            sems.at[k],
        )
        c.start()
        copies.append(c)

    r0.wait()
    rl.wait()
    col_ids = jax.lax.broadcasted_iota(jnp.int32, (1, n_cols), 1)
    row0[...] = jnp.where(col_ids == n_cols - 2, 1.0, row0[...])
    rowl[...] = jnp.where(col_ids == 1, 5.0, rowl[...])

    for c in copies:
        c.wait()

    w0 = pltpu.make_async_copy(row0, o_ref.at[0:1, :], row_sem)
    w0.start()
    wl = pltpu.make_async_copy(rowl, o_ref.at[n_rows - 1 : n_rows, :], row_sem)
    wl.start()
    w0.wait()
    wl.wait()


@jax.jit
def kernel(x):
    n_rows, n_cols = x.shape
    return pl.pallas_call(
        _copy_scatter_kernel,
        in_specs=[pl.BlockSpec(memory_space=pl.ANY)],
        out_specs=pl.BlockSpec(memory_space=pl.ANY),
        out_shape=jax.ShapeDtypeStruct(x.shape, x.dtype),
        scratch_shapes=[
            pltpu.VMEM((1, n_cols), x.dtype),
            pltpu.VMEM((1, n_cols), x.dtype),
            pltpu.SemaphoreType.DMA((_N_CHUNKS,)),
            pltpu.SemaphoreType.DMA,
        ],
    )(x)


# speedup vs baseline: 48.4667x; 48.4667x over previous
import jax
import jax.numpy as jnp
from jax.experimental import pallas as pl
from jax.experimental.pallas import tpu as pltpu

_BLOCK_ROWS = 256


def _copy_scatter_kernel(x_ref, o_ref):
    o_ref[...] = x_ref[...]
    i = pl.program_id(0)
    n = pl.num_programs(0)
    n_cols = o_ref.shape[1]
    col_ids = jax.lax.broadcasted_iota(jnp.int32, (1, n_cols), 1)

    @pl.when(i == 0)
    def _():
        o_ref[0:1, :] = jnp.where(col_ids == n_cols - 2, 1.0, x_ref[0:1, :])

    @pl.when(i == n - 1)
    def _():
        last = o_ref.shape[0] - 1
        o_ref[last : last + 1, :] = jnp.where(
            col_ids == 1, 5.0, x_ref[last : last + 1, :]
        )


@jax.jit
def kernel(x):
    n_rows, n_cols = x.shape
    grid = n_rows // _BLOCK_ROWS
    return pl.pallas_call(
        _copy_scatter_kernel,
        grid=(grid,),
        in_specs=[pl.BlockSpec((_BLOCK_ROWS, n_cols), lambda i: (i, 0))],
        out_specs=pl.BlockSpec((_BLOCK_ROWS, n_cols), lambda i: (i, 0)),
        out_shape=jax.ShapeDtypeStruct(x.shape, x.dtype),
        compiler_params=pltpu.CompilerParams(
            dimension_semantics=("parallel",),
        ),
    )(x)
